# initial kernel scaffold (unmeasured)
import jax
import jax.numpy as jnp
from jax import lax
from jax.experimental import pallas as pl
from jax.experimental.pallas import tpu as pltpu

N_DEV = 16


def _gelu(y):
    c = 0.7978845608028654
    return 0.5 * y * (1.0 + jnp.tanh(c * (y + 0.044715 * y * y * y)))


def kernel(x, w_mat):
    m_per, k = x.shape
    _, n_per = w_mat.shape
    m_total = N_DEV * m_per

    def body(x_ref, w_ref, out_ref, comm_ref, send_sems, recv_sems):
        my = lax.axis_index("i")
        left = (my + N_DEV - 1) % N_DEV
        right = (my + 1) % N_DEV

        barrier_sem = pltpu.get_barrier_semaphore()
        for nbr in (left, right):
            pl.semaphore_signal(
                barrier_sem, inc=1,
                device_id=(nbr,), device_id_type=pl.DeviceIdType.MESH,
            )
        pl.semaphore_wait(barrier_sem, 2)

        def compute(o):
            rows = comm_ref[pl.ds(o * m_per, m_per), :]
            acc = jnp.dot(rows, w_ref[...], preferred_element_type=jnp.float32)
            out_ref[pl.ds(o * m_per, m_per), :] = _gelu(acc)

        comm_ref[pl.ds(my * m_per, m_per), :] = x_ref[...]

        for h in range(N_DEV - 1):
            s = (my + N_DEV - h) % N_DEV
            r = (my + N_DEV - 1 - h) % N_DEV
            send = pltpu.make_async_remote_copy(
                src_ref=comm_ref.at[pl.ds(s * m_per, m_per)],
                dst_ref=comm_ref.at[pl.ds(s * m_per, m_per)],
                send_sem=send_sems.at[s],
                recv_sem=recv_sems.at[s],
                device_id=(right,),
                device_id_type=pl.DeviceIdType.MESH,
            )
            send.start()
            recv = pltpu.make_async_remote_copy(
                src_ref=comm_ref.at[pl.ds(r * m_per, m_per)],
                dst_ref=comm_ref.at[pl.ds(r * m_per, m_per)],
                send_sem=send_sems.at[r],
                recv_sem=recv_sems.at[r],
                device_id=(left,),
                device_id_type=pl.DeviceIdType.MESH,
            )
            if h == 0:
                compute(my)
            recv.wait_recv()
            compute(r)
            send.wait_send()

    return pl.pallas_call(
        body,
        out_shape=jax.ShapeDtypeStruct((m_total, n_per), jnp.float32),
        in_specs=[
            pl.BlockSpec(memory_space=pltpu.VMEM),
            pl.BlockSpec(memory_space=pltpu.VMEM),
        ],
        out_specs=pl.BlockSpec(memory_space=pltpu.VMEM),
        scratch_shapes=[
            pltpu.VMEM((m_total, k), x.dtype),
            pltpu.SemaphoreType.DMA((N_DEV,)),
            pltpu.SemaphoreType.DMA((N_DEV,)),
        ],
        compiler_params=pltpu.CompilerParams(collective_id=0),
    )(x, w_mat)


# baseline (device time: 409930 ns/iter reference)
import jax
import jax.numpy as jnp
from jax import lax
from jax.experimental import pallas as pl
from jax.experimental.pallas import tpu as pltpu

N_DEV = 16


def _gelu(y):
    c = 0.7978845608028654
    return 0.5 * y * (1.0 + jnp.tanh(c * (y + 0.044715 * y * y * y)))


def kernel(x, w_mat):
    x = x.astype(jnp.bfloat16)
    w_mat = w_mat.astype(jnp.bfloat16)
    m_per, k = x.shape
    _, n_per = w_mat.shape
    m_total = N_DEV * m_per

    def body(x_ref, w_ref, out_ref, comm_ref, send_sems, recv_sems):
        my = lax.axis_index("i")
        left = (my + N_DEV - 1) % N_DEV
        right = (my + 1) % N_DEV

        barrier_sem = pltpu.get_barrier_semaphore()
        for nbr in (left, right):
            pl.semaphore_signal(
                barrier_sem, inc=1,
                device_id=(nbr,), device_id_type=pl.DeviceIdType.MESH,
            )
        pl.semaphore_wait(barrier_sem, 2)

        def compute(o):
            rows = comm_ref[pl.ds(o * m_per, m_per), :]
            acc = jnp.dot(rows, w_ref[...], preferred_element_type=jnp.float32)
            out_ref[pl.ds(o * m_per, m_per), :] = _gelu(acc)

        comm_ref[pl.ds(my * m_per, m_per), :] = x_ref[...]

        for h in range(N_DEV - 1):
            s = (my + N_DEV - h) % N_DEV
            r = (my + N_DEV - 1 - h) % N_DEV
            send = pltpu.make_async_remote_copy(
                src_ref=comm_ref.at[pl.ds(s * m_per, m_per)],
                dst_ref=comm_ref.at[pl.ds(s * m_per, m_per)],
                send_sem=send_sems.at[s],
                recv_sem=recv_sems.at[s],
                device_id=(right,),
                device_id_type=pl.DeviceIdType.MESH,
            )
            send.start()
            recv = pltpu.make_async_remote_copy(
                src_ref=comm_ref.at[pl.ds(r * m_per, m_per)],
                dst_ref=comm_ref.at[pl.ds(r * m_per, m_per)],
                send_sem=send_sems.at[r],
                recv_sem=recv_sems.at[r],
                device_id=(left,),
                device_id_type=pl.DeviceIdType.MESH,
            )
            if h == 0:
                compute(my)
            recv.wait_recv()
            compute(r)
            send.wait_send()

    return pl.pallas_call(
        body,
        out_shape=jax.ShapeDtypeStruct((m_total, n_per), jnp.float32),
        in_specs=[
            pl.BlockSpec(memory_space=pltpu.VMEM),
            pl.BlockSpec(memory_space=pltpu.VMEM),
        ],
        out_specs=pl.BlockSpec(memory_space=pltpu.VMEM),
        scratch_shapes=[
            pltpu.VMEM((m_total, k), x.dtype),
            pltpu.SemaphoreType.DMA((N_DEV,)),
            pltpu.SemaphoreType.DMA((N_DEV,)),
        ],
        compiler_params=pltpu.CompilerParams(
            collective_id=0,
            vmem_limit_bytes=48 * 1024 * 1024,
        ),
    )(x, w_mat)


# device time: 219933 ns/iter; 1.8639x vs baseline; 1.8639x over previous
import jax
import jax.numpy as jnp
from jax import lax
from jax.experimental import pallas as pl
from jax.experimental.pallas import tpu as pltpu

N_DEV = 16
R_HOPS = N_DEV // 2
L_HOPS = N_DEV - 1 - R_HOPS


def _gelu(y):
    c = 0.7978845608028654
    return 0.5 * y * (1.0 + jnp.tanh(c * (y + 0.044715 * y * y * y)))


def kernel(x, w_mat):
    x = x.astype(jnp.bfloat16)
    w_mat = w_mat.astype(jnp.bfloat16)
    m_per, k = x.shape
    _, n_per = w_mat.shape
    m_total = N_DEV * m_per

    def body(x_ref, w_ref, out_ref, comm_ref, ssem_r, ssem_l, rsem):
        my = lax.axis_index("i")
        left = (my + N_DEV - 1) % N_DEV
        right = (my + 1) % N_DEV

        barrier_sem = pltpu.get_barrier_semaphore()
        for nbr in (left, right):
            pl.semaphore_signal(
                barrier_sem, inc=1,
                device_id=(nbr,), device_id_type=pl.DeviceIdType.MESH,
            )
        pl.semaphore_wait(barrier_sem, 2)

        def slot(o):
            return pl.ds((o % N_DEV) * m_per, m_per)

        def mk(o, sem, dev):
            s = slot(o)
            return pltpu.make_async_remote_copy(
                src_ref=comm_ref.at[s],
                dst_ref=comm_ref.at[s],
                send_sem=sem,
                recv_sem=rsem.at[(o % N_DEV) * 1],
                device_id=(dev,),
                device_id_type=pl.DeviceIdType.MESH,
            )

        def compute(o):
            rows = comm_ref[slot(o), :]
            acc = jnp.dot(rows, w_ref[...], preferred_element_type=jnp.float32)
            out_ref[slot(o), :] = _gelu(acc)

        comm_ref[slot(my), :] = x_ref[...]
        sends = []
        sr = mk(my, ssem_r.at[0], right)
        sr.start()
        sl = mk(my, ssem_l.at[0], left)
        sl.start()
        sends += [sr, sl]
        compute(my)

        for h in range(R_HOPS):
            r = (my + N_DEV - 1 - h) % N_DEV
            mk(r, ssem_r.at[0], left).wait_recv()
            if h + 1 < R_HOPS:
                s = mk(r, ssem_r.at[h + 1], right)
                s.start()
                sends.append(s)
            compute(r)
            if h < L_HOPS:
                r2 = (my + 1 + h) % N_DEV
                mk(r2, ssem_l.at[0], right).wait_recv()
                if h + 1 < L_HOPS:
                    s = mk(r2, ssem_l.at[h + 1], left)
                    s.start()
                    sends.append(s)
                compute(r2)

        for s in sends:
            s.wait_send()

    return pl.pallas_call(
        body,
        out_shape=jax.ShapeDtypeStruct((m_total, n_per), jnp.float32),
        in_specs=[
            pl.BlockSpec(memory_space=pltpu.VMEM),
            pl.BlockSpec(memory_space=pltpu.VMEM),
        ],
        out_specs=pl.BlockSpec(memory_space=pltpu.VMEM),
        scratch_shapes=[
            pltpu.VMEM((m_total, k), x.dtype),
            pltpu.SemaphoreType.DMA((R_HOPS,)),
            pltpu.SemaphoreType.DMA((L_HOPS,)),
            pltpu.SemaphoreType.DMA((N_DEV,)),
        ],
        compiler_params=pltpu.CompilerParams(
            collective_id=0,
            vmem_limit_bytes=48 * 1024 * 1024,
        ),
    )(x, w_mat)


# device time: 218468 ns/iter; 1.8764x vs baseline; 1.0067x over previous
import jax
import jax.numpy as jnp
from jax import lax
from jax.experimental import pallas as pl
from jax.experimental.pallas import tpu as pltpu

N_DEV = 16
R_HOPS = N_DEV // 2
L_HOPS = N_DEV - 1 - R_HOPS


def _gelu(y):
    c = 0.7978845608028654
    return 0.5 * y * (1.0 + jnp.tanh(c * (y + 0.044715 * y * y * y)))


def kernel(x, w_mat):
    x = x.astype(jnp.bfloat16)
    w_mat = w_mat.astype(jnp.bfloat16)
    m_per, k = x.shape
    _, n_per = w_mat.shape
    m_total = N_DEV * m_per

    def body(x_ref, w_ref, out_ref, comm_ref, ssem_r, ssem_l, rsem):
        my = lax.axis_index("i")
        left = (my + N_DEV - 1) % N_DEV
        right = (my + 1) % N_DEV

        barrier_sem = pltpu.get_barrier_semaphore()
        for nbr in (left, right):
            pl.semaphore_signal(
                barrier_sem, inc=1,
                device_id=(nbr,), device_id_type=pl.DeviceIdType.MESH,
            )
        pl.semaphore_wait(barrier_sem, 2)

        def slot(o):
            return pl.ds((o % N_DEV) * m_per, m_per)

        def mk(o, sem, dev, src=None):
            s = slot(o)
            return pltpu.make_async_remote_copy(
                src_ref=comm_ref.at[s] if src is None else src,
                dst_ref=comm_ref.at[s],
                send_sem=sem,
                recv_sem=rsem.at[(o % N_DEV) * 1],
                device_id=(dev,),
                device_id_type=pl.DeviceIdType.MESH,
            )

        def compute(o):
            rows = comm_ref[slot(o), :]
            acc = jnp.dot(rows, w_ref[...], preferred_element_type=jnp.float32)
            out_ref[slot(o), :] = _gelu(acc)

        sends = []
        sr = mk(my, ssem_r.at[0], right, src=x_ref)
        sr.start()
        sl = mk(my, ssem_l.at[0], left, src=x_ref)
        sl.start()
        sends += [sr, sl]
        comm_ref[slot(my), :] = x_ref[...]
        compute(my)

        for h in range(R_HOPS):
            r = (my + N_DEV - 1 - h) % N_DEV
            r2 = (my + 1 + h) % N_DEV
            mk(r, ssem_r.at[0], left).wait_recv()
            if h + 1 < R_HOPS:
                s = mk(r, ssem_r.at[h + 1], right)
                s.start()
                sends.append(s)
            if h < L_HOPS:
                mk(r2, ssem_l.at[0], right).wait_recv()
                if h + 1 < L_HOPS:
                    s = mk(r2, ssem_l.at[h + 1], left)
                    s.start()
                    sends.append(s)
            compute(r)
            if h < L_HOPS:
                compute(r2)

        for s in sends:
            s.wait_send()

    return pl.pallas_call(
        body,
        out_shape=jax.ShapeDtypeStruct((m_total, n_per), jnp.float32),
        in_specs=[
            pl.BlockSpec(memory_space=pltpu.VMEM),
            pl.BlockSpec(memory_space=pltpu.VMEM),
        ],
        out_specs=pl.BlockSpec(memory_space=pltpu.VMEM),
        scratch_shapes=[
            pltpu.VMEM((m_total, k), x.dtype),
            pltpu.SemaphoreType.DMA((R_HOPS,)),
            pltpu.SemaphoreType.DMA((L_HOPS,)),
            pltpu.SemaphoreType.DMA((N_DEV,)),
        ],
        compiler_params=pltpu.CompilerParams(
            collective_id=0,
            vmem_limit_bytes=48 * 1024 * 1024,
        ),
    )(x, w_mat)


# device time: 209555 ns/iter; 1.9562x vs baseline; 1.0425x over previous
import jax
import jax.numpy as jnp
from jax import lax
from jax.experimental import pallas as pl
from jax.experimental.pallas import tpu as pltpu

N_DEV = 16
HOPS = N_DEV // 2


def _gelu(y):
    c = 0.7978845608028654
    return 0.5 * y * (1.0 + jnp.tanh(c * (y + 0.044715 * y * y * y)))


def kernel(x, w_mat):
    x = x.astype(jnp.bfloat16)
    w_mat = w_mat.astype(jnp.bfloat16)
    m_per, k = x.shape
    _, n_per = w_mat.shape
    m_total = N_DEV * m_per
    m_half = m_per // 2

    def body(x_ref, w_ref, out_ref, comm_ref, ssem_r, ssem_l, rsem, rsem_half):
        my = lax.axis_index("i")
        left = (my + N_DEV - 1) % N_DEV
        right = (my + 1) % N_DEV

        barrier_sem = pltpu.get_barrier_semaphore()
        for nbr in (left, right):
            pl.semaphore_signal(
                barrier_sem, inc=1,
                device_id=(nbr,), device_id_type=pl.DeviceIdType.MESH,
            )
        pl.semaphore_wait(barrier_sem, 2)

        def rows(o, lo, sz):
            return pl.ds((o % N_DEV) * m_per + lo, sz)

        def mk(o, lo, sz, sem, recv, dev, src=None):
            return pltpu.make_async_remote_copy(
                src_ref=comm_ref.at[rows(o, lo, sz)] if src is None else src,
                dst_ref=comm_ref.at[rows(o, lo, sz)],
                send_sem=sem,
                recv_sem=recv,
                device_id=(dev,),
                device_id_type=pl.DeviceIdType.MESH,
            )

        def compute(o):
            chunk = comm_ref[rows(o, 0, m_per), :]
            acc = jnp.dot(chunk, w_ref[...], preferred_element_type=jnp.float32)
            out_ref[rows(o, 0, m_per), :] = _gelu(acc)

        sends = []
        s = mk(my, 0, m_per, ssem_r.at[0], rsem.at[my % N_DEV], right, src=x_ref)
        s.start()
        sends.append(s)
        s = mk(my, 0, m_per, ssem_l.at[0], rsem.at[my % N_DEV], left, src=x_ref)
        s.start()
        sends.append(s)
        comm_ref[rows(my, 0, m_per), :] = x_ref[...]
        compute(my)

        for h in range(HOPS):
            r = (my + N_DEV - 1 - h) % N_DEV
            r2 = (my + 1 + h) % N_DEV
            if h < HOPS - 1:
                mk(r, 0, m_per, ssem_r.at[0], rsem.at[r % N_DEV], left).wait_recv()
                if h < HOPS - 2:
                    s = mk(r, 0, m_per, ssem_r.at[h + 1], rsem.at[r % N_DEV], right)
                else:
                    s = mk(r, 0, m_half, ssem_r.at[h + 1], rsem.at[r % N_DEV], right)
                s.start()
                sends.append(s)
                mk(r2, 0, m_per, ssem_l.at[0], rsem.at[r2 % N_DEV], right).wait_recv()
                if h < HOPS - 2:
                    s = mk(r2, 0, m_per, ssem_l.at[h + 1], rsem.at[r2 % N_DEV], left)
                else:
                    s = mk(r2, m_half, m_half, ssem_l.at[h + 1], rsem_half.at[0], left)
                s.start()
                sends.append(s)
                compute(r)
                compute(r2)
            else:
                mk(r, 0, m_half, ssem_r.at[0], rsem.at[r % N_DEV], left).wait_recv()
                mk(r, m_half, m_half, ssem_l.at[0], rsem_half.at[0], right).wait_recv()
                compute(r)

        for s in sends:
            s.wait_send()

    return pl.pallas_call(
        body,
        out_shape=jax.ShapeDtypeStruct((m_total, n_per), jnp.float32),
        in_specs=[
            pl.BlockSpec(memory_space=pltpu.VMEM),
            pl.BlockSpec(memory_space=pltpu.VMEM),
        ],
        out_specs=pl.BlockSpec(memory_space=pltpu.VMEM),
        scratch_shapes=[
            pltpu.VMEM((m_total, k), x.dtype),
            pltpu.SemaphoreType.DMA((HOPS,)),
            pltpu.SemaphoreType.DMA((HOPS,)),
            pltpu.SemaphoreType.DMA((N_DEV,)),
            pltpu.SemaphoreType.DMA((1,)),
        ],
        compiler_params=pltpu.CompilerParams(
            collective_id=0,
            vmem_limit_bytes=48 * 1024 * 1024,
        ),
    )(x, w_mat)


# device time: 208965 ns/iter; 1.9617x vs baseline; 1.0028x over previous
import jax
import jax.numpy as jnp
from jax import lax
from jax.experimental import pallas as pl
from jax.experimental.pallas import tpu as pltpu

N_DEV = 16
HOPS = N_DEV // 2


def _gelu(y):
    c = 0.7978845608028654
    return 0.5 * y * (1.0 + jnp.tanh(c * (y + 0.044715 * y * y * y)))


def kernel(x, w_mat):
    x = x.astype(jnp.bfloat16)
    w_mat = w_mat.astype(jnp.bfloat16)
    m_per, k = x.shape
    _, n_per = w_mat.shape
    m_total = N_DEV * m_per
    m_half = m_per // 2

    def body(x_ref, w_ref, out_ref, comm_ref, ssem_r, ssem_l, rsem, rsem_half):
        my = lax.axis_index("i")
        left = (my + N_DEV - 1) % N_DEV
        right = (my + 1) % N_DEV

        barrier_sem = pltpu.get_barrier_semaphore()
        for nbr in (left, right):
            pl.semaphore_signal(
                barrier_sem, inc=1,
                device_id=(nbr,), device_id_type=pl.DeviceIdType.MESH,
            )
        pl.semaphore_wait(barrier_sem, 2)

        def rows(o, lo, sz):
            return pl.ds((o % N_DEV) * m_per + lo, sz)

        def mk(o, lo, sz, sem, recv, dev, src=None):
            return pltpu.make_async_remote_copy(
                src_ref=comm_ref.at[rows(o, lo, sz)] if src is None else src,
                dst_ref=comm_ref.at[rows(o, lo, sz)],
                send_sem=sem,
                recv_sem=recv,
                device_id=(dev,),
                device_id_type=pl.DeviceIdType.MESH,
            )

        def compute(o, lo=0, sz=None):
            sz = m_per if sz is None else sz
            chunk = comm_ref[rows(o, lo, sz), :]
            acc = jnp.dot(chunk, w_ref[...], preferred_element_type=jnp.float32)
            out_ref[rows(o, lo, sz), :] = _gelu(acc)

        sends = []
        s = mk(my, 0, m_per, ssem_r.at[0], rsem.at[my % N_DEV], right, src=x_ref)
        s.start()
        sends.append(s)
        s = mk(my, 0, m_per, ssem_l.at[0], rsem.at[my % N_DEV], left, src=x_ref)
        s.start()
        sends.append(s)
        comm_ref[rows(my, 0, m_per), :] = x_ref[...]
        compute(my)

        for h in range(HOPS):
            r = (my + N_DEV - 1 - h) % N_DEV
            r2 = (my + 1 + h) % N_DEV
            if h < HOPS - 1:
                mk(r, 0, m_per, ssem_r.at[0], rsem.at[r % N_DEV], left).wait_recv()
                if h < HOPS - 2:
                    s = mk(r, 0, m_per, ssem_r.at[h + 1], rsem.at[r % N_DEV], right)
                else:
                    s = mk(r, 0, m_half, ssem_r.at[h + 1], rsem.at[r % N_DEV], right)
                s.start()
                sends.append(s)
                mk(r2, 0, m_per, ssem_l.at[0], rsem.at[r2 % N_DEV], right).wait_recv()
                if h < HOPS - 2:
                    s = mk(r2, 0, m_per, ssem_l.at[h + 1], rsem.at[r2 % N_DEV], left)
                else:
                    s = mk(r2, m_half, m_half, ssem_l.at[h + 1], rsem_half.at[0], left)
                s.start()
                sends.append(s)
                compute(r)
                compute(r2)
            else:
                mk(r, 0, m_half, ssem_r.at[0], rsem.at[r % N_DEV], left).wait_recv()
                compute(r, 0, m_half)
                mk(r, m_half, m_half, ssem_l.at[0], rsem_half.at[0], right).wait_recv()
                compute(r, m_half, m_half)

        for s in sends:
            s.wait_send()

    return pl.pallas_call(
        body,
        out_shape=jax.ShapeDtypeStruct((m_total, n_per), jnp.float32),
        in_specs=[
            pl.BlockSpec(memory_space=pltpu.VMEM),
            pl.BlockSpec(memory_space=pltpu.VMEM),
        ],
        out_specs=pl.BlockSpec(memory_space=pltpu.VMEM),
        scratch_shapes=[
            pltpu.VMEM((m_total, k), x.dtype),
            pltpu.SemaphoreType.DMA((HOPS,)),
            pltpu.SemaphoreType.DMA((HOPS,)),
            pltpu.SemaphoreType.DMA((N_DEV,)),
            pltpu.SemaphoreType.DMA((1,)),
        ],
        compiler_params=pltpu.CompilerParams(
            collective_id=0,
            vmem_limit_bytes=48 * 1024 * 1024,
        ),
    )(x, w_mat)
